# baseline (speedup 1.0000x reference)
import functools

import jax
import jax.numpy as jnp
from jax import lax
from jax.experimental import pallas as pl
from jax.experimental.pallas import tpu as pltpu
from jax.experimental.pallas import tpu_sc as plsc

N = 10000
E = 320000
D = 128
H = 64
NG = 32
NCLS = 10

NCORE = 2
NSUB = 16
NW = NCORE * NSUB
EPW = E // NW
CHUNK = 128
CPW = 80
SCE = CHUNK
NSC = CPW
NPASS = 4
RPP = CPW // NPASS
NBUF = 2
PD = 1
YRT = N // NSUB
EPW_PAD = CPW * CHUNK
NPAD = 10240
RPT = NPAD // NSUB
ZCH = 128
CNTW = 8

BLK = 2000
BLK2 = BLK // 2
GRID = N // BLK


def _sc_body(with_counts, *refs):
    r = list(refs)
    if with_counts:
        (y_s, y_f, src_s, dst_s, src_f, dst_f, zeros_h, zcnt_h, ones_h,
         out_s, out_f, ocnt_s, ocnt_f, acc, cnt, y_sh) = r[:16]
        r = r[16:]
    else:
        (y_s, y_f, src_s, dst_s, src_f, dst_f, zeros_h,
         out_s, out_f, acc, y_sh) = r[:11]
        r = r[11:]
        cnt = ocnt_s = ocnt_f = None
        zcnt_h = ones_h = None
    src_v, dst_v = r[0], r[1]
    bufs = tuple(r[2:2 + NBUF])
    r = r[2 + NBUF:]
    if with_counts:
        ones_v = r[0]
        r = r[1:]
    else:
        ones_v = None
    gsems = tuple(r[:NBUF])
    ssems = tuple(r[NBUF:2 * NBUF])

    c = lax.axis_index("c")
    s = lax.axis_index("s")
    w = c * NSUB + s

    if with_counts:
        pltpu.sync_copy(ones_h, ones_v)

    branches = [(y_s, src_s, dst_s, out_s, ocnt_s if with_counts else None),
                (y_f, src_f, dst_f, out_f, ocnt_f if with_counts else None)]
    for y_h, src_h, dst_h, out_h, ocnt_h in branches:
        pltpu.sync_copy(y_h.at[pl.ds(s * YRT, YRT)],
                        y_sh.at[pl.ds(s * YRT, YRT)])
        for k in range(RPT // ZCH):
            r0 = s * RPT + k * ZCH
            pltpu.sync_copy(zeros_h, acc.at[pl.ds(r0, ZCH)])
            if with_counts:
                pltpu.sync_copy(zcnt_h, cnt.at[pl.ds(r0, ZCH)])
        plsc.subcore_barrier()

        def gstart(g, i):
            pltpu.async_copy(y_sh.at[src_v.at[g]], bufs[i], gsems[i])

        def gwait(g, i):
            pltpu.make_async_copy(y_sh.at[src_v.at[g]], bufs[i],
                                  gsems[i]).wait()

        def sstart(g, i):
            pltpu.async_copy(bufs[i], acc.at[dst_v.at[g]], ssems[i],
                             add=True)
            if with_counts:
                pltpu.async_copy(ones_v, cnt.at[dst_v.at[g]], ssems[i],
                                 add=True)

        def swait(g, i):
            pltpu.make_async_copy(bufs[i], acc.at[dst_v.at[g]],
                                  ssems[i]).wait()
            if with_counts:
                pltpu.make_async_copy(ones_v, cnt.at[dst_v.at[g]],
                                      ssems[i]).wait()

        for p in range(NPASS):
            base = w * CPW + p * RPP
            pltpu.sync_copy(src_h.at[pl.ds(base, RPP)], src_v)
            pltpu.sync_copy(dst_h.at[pl.ds(base, RPP)], dst_v)

            for k in range(PD):
                gstart(k, k % NBUF)

            def quad(q, carry):
                for i in range(NBUF):
                    g = NBUF * q + i
                    gwait(g, i)
                    sstart(g, i)
                    j = (i + PD) % NBUF

                    @pl.when(g >= PD)
                    def _():
                        swait(g - PD, j)

                    @pl.when(g + PD < RPP)
                    def _():
                        gstart(g + PD, j)
                return carry

            lax.fori_loop(0, RPP // NBUF, quad, 0)
            for t in range(PD):
                g = RPP - PD + t
                swait(g, g % NBUF)
        plsc.subcore_barrier()

        r0 = s * RPT
        pltpu.sync_copy(acc.at[pl.ds(r0, RPT)], out_h.at[c, pl.ds(r0, RPT)])
        if with_counts:
            pltpu.sync_copy(cnt.at[pl.ds(r0, RPT)],
                            ocnt_h.at[c, pl.ds(r0, RPT)])


def _make_sc_kernel(with_counts):
    out_type = [jax.ShapeDtypeStruct((NCORE, NPAD, H), jnp.float32),
                jax.ShapeDtypeStruct((NCORE, NPAD, H), jnp.float32)]
    scratch = [pltpu.VMEM_SHARED((NPAD, H), jnp.float32)]
    if with_counts:
        out_type += [jax.ShapeDtypeStruct((NCORE, NPAD, CNTW), jnp.float32),
                     jax.ShapeDtypeStruct((NCORE, NPAD, CNTW), jnp.float32)]
        scratch += [pltpu.VMEM_SHARED((NPAD, CNTW), jnp.float32)]
    scratch += [pltpu.VMEM_SHARED((N, H), jnp.float32)]
    scratch += [pltpu.VMEM((RPP, CHUNK), jnp.int32),
                pltpu.VMEM((RPP, CHUNK), jnp.int32)]
    scratch += [pltpu.VMEM((SCE, H), jnp.float32)] * NBUF
    if with_counts:
        scratch += [pltpu.VMEM((SCE, CNTW), jnp.float32)]
    scratch += [pltpu.SemaphoreType.DMA] * (2 * NBUF)
    mesh = plsc.VectorSubcoreMesh(core_axis_name="c", subcore_axis_name="s")
    return pl.kernel(functools.partial(_sc_body, with_counts),
                     out_type=out_type, mesh=mesh, scratch_types=scratch,
                     compiler_params=pltpu.CompilerParams(
                         use_tc_tiling_on_sc=False))


def _pair_mm(xe, xo, w):
    return jnp.concatenate(
        [jnp.dot(xe, w, preferred_element_type=jnp.float32),
         jnp.dot(xo, w, preferred_element_type=jnp.float32)], axis=1)


def _t0_body(xp, fxp, ws_rel, ws_root, wf_rel, wf_root,
             y1s, r1s, y1f, r1f):
    xb = xp[...]
    fb = fxp[...]
    xe, xo = xb[:, :D], xb[:, D:]
    fe, fo = fb[:, :D], fb[:, D:]
    y1s[...] = _pair_mm(xe, xo, ws_rel[...])
    r1s[...] = _pair_mm(xe, xo, ws_root[...])
    y1f[...] = _pair_mm(fe, fo, wf_rel[...])
    r1f[...] = _pair_mm(fe, fo, wf_root[...])


def _t0(xp, fxp, ws_rel, ws_root, wf_rel, wf_root):
    blk = pl.BlockSpec((BLK2, 2 * D), lambda i: (i, 0))
    wspec = pl.BlockSpec((D, H), lambda i: (0, 0))
    ospec = pl.BlockSpec((BLK2, 2 * H), lambda i: (i, 0))
    return pl.pallas_call(
        _t0_body,
        grid=(GRID,),
        in_specs=[blk, blk, wspec, wspec, wspec, wspec],
        out_specs=[ospec] * 4,
        out_shape=[jax.ShapeDtypeStruct((N // 2, 2 * H), jnp.float32)] * 4,
    )(xp, fxp, ws_rel, ws_root, wf_rel, wf_root)


def _conv_out(parts, cnt, r, b):
    tot = parts[0] + parts[1]
    c2 = cnt[0] + cnt[1]
    ce = jnp.maximum(c2[:, 0:1], 1.0)
    co = jnp.maximum(c2[:, CNTW:CNTW + 1], 1.0)
    div = jnp.concatenate([jnp.broadcast_to(ce, (BLK2, H)),
                           jnp.broadcast_to(co, (BLK2, H))], axis=1)
    return jax.nn.relu(tot / div + b[...] + r[...])


def _onehot(bids):
    gids = lax.broadcasted_iota(jnp.int32, (BLK2, NG), 1)
    return (bids[:, None] == gids).astype(jnp.float32)


def _pool_contrib(batch_ref, cols_e, cols_o):
    ohe = _onehot(batch_ref[0, 0, 0, :])
    oho = _onehot(batch_ref[1, 0, 0, :])
    dims = (((0,), (0,)), ((), ()))
    aug_e = jnp.concatenate(cols_e, axis=1)
    aug_o = jnp.concatenate(cols_o, axis=1)
    return (lax.dot_general(ohe, aug_e, dims,
                            preferred_element_type=jnp.float32)
            + lax.dot_general(oho, aug_o, dims,
                              preferred_element_type=jnp.float32))


def _t1_body(ps, cs, rs, bs, w2s_rel, w2s_root,
             pf, cf, rf, bf, w2f_rel, w2f_root, batch,
             y2s, r2s, y2f, r2f, pool1):
    i = pl.program_id(0)
    hs = _conv_out(ps[...], cs[...], rs, bs)
    hf = _conv_out(pf[...], cf[...], rf, bf)
    hse, hso = hs[:, :H], hs[:, H:]
    hfe, hfo = hf[:, :H], hf[:, H:]
    y2s[...] = _pair_mm(hse, hso, w2s_rel[...])
    r2s[...] = _pair_mm(hse, hso, w2s_root[...])
    y2f[...] = _pair_mm(hfe, hfo, w2f_rel[...])
    r2f[...] = _pair_mm(hfe, hfo, w2f_root[...])
    ones = jnp.ones((BLK2, H), jnp.float32)
    contrib = _pool_contrib(batch, [hse, hfe, ones], [hso, hfo, ones])

    @pl.when(i == 0)
    def _():
        pool1[...] = jnp.zeros_like(pool1)

    pool1[...] += contrib


def _t1(ps, cs, rs, bs, w2s_rel, w2s_root, pf, cf, rf, bf, w2f_rel, w2f_root,
        batch3):
    pspec = pl.BlockSpec((NCORE, BLK2, 2 * H), lambda i: (0, i, 0))
    cspec = pl.BlockSpec((NCORE, BLK2, 2 * CNTW), lambda i: (0, i, 0))
    rspec = pl.BlockSpec((BLK2, 2 * H), lambda i: (i, 0))
    bspec = pl.BlockSpec((1, 2 * H), lambda i: (0, 0))
    wspec = pl.BlockSpec((H, H), lambda i: (0, 0))
    ospec = pl.BlockSpec((BLK2, 2 * H), lambda i: (i, 0))
    batspec = pl.BlockSpec((2, 1, 1, BLK2), lambda i: (0, i, 0, 0))
    pool_spec = pl.BlockSpec((NG, 3 * H), lambda i: (0, 0))
    return pl.pallas_call(
        _t1_body,
        grid=(GRID,),
        in_specs=[pspec, cspec, rspec, bspec, wspec, wspec,
                  pspec, cspec, rspec, bspec, wspec, wspec, batspec],
        out_specs=[ospec] * 4 + [pool_spec],
        out_shape=[jax.ShapeDtypeStruct((N // 2, 2 * H), jnp.float32)] * 4
        + [jax.ShapeDtypeStruct((NG, 3 * H), jnp.float32)],
    )(ps, cs, rs, bs, w2s_rel, w2s_root, pf, cf, rf, bf, w2f_rel, w2f_root,
      batch3)


def _t2_body(ps, cs, rs, bs, pf, cf, rf, bf, batch, pool1,
             w1, b1, w2, b2, w3, b3, out, pool2):
    i = pl.program_id(0)
    h2s = _conv_out(ps[...], cs[...], rs, bs)
    h2f = _conv_out(pf[...], cf[...], rf, bf)
    contrib = _pool_contrib(batch, [h2s[:, :H], h2f[:, :H]],
                            [h2s[:, H:], h2f[:, H:]])

    @pl.when(i == 0)
    def _():
        pool2[...] = jnp.zeros_like(pool2)

    pool2[...] += contrib

    @pl.when(i == GRID - 1)
    def _():
        p1 = pool1[...]
        p2 = pool2[...]
        gc = jnp.maximum(p1[:, 2 * H:2 * H + 1], 1.0)
        z = jnp.concatenate([p1[:, :H] / gc, p2[:, :H] / gc,
                             p1[:, H:2 * H], p2[:, H:2 * H]], axis=1)
        z = jax.nn.relu(
            jnp.dot(z, w1[...], preferred_element_type=jnp.float32) + b1[...])
        z = jax.nn.relu(
            jnp.dot(z, w2[...], preferred_element_type=jnp.float32) + b2[...])
        z = jnp.dot(z, w3[...], preferred_element_type=jnp.float32) + b3[...]
        m = jnp.max(z, axis=1, keepdims=True)
        lse = m + jnp.log(jnp.sum(jnp.exp(z - m), axis=1, keepdims=True))
        out[...] = z - lse


def _t2(ps, cs, rs, bs, pf, cf, rf, bf, batch3, pool1, w1, b1, w2, b2,
        w3, b3):
    pspec = pl.BlockSpec((NCORE, BLK2, 2 * H), lambda i: (0, i, 0))
    cspec = pl.BlockSpec((NCORE, BLK2, 2 * CNTW), lambda i: (0, i, 0))
    rspec = pl.BlockSpec((BLK2, 2 * H), lambda i: (i, 0))
    bspec = pl.BlockSpec((1, 2 * H), lambda i: (0, 0))
    batspec = pl.BlockSpec((2, 1, 1, BLK2), lambda i: (0, i, 0, 0))
    full = lambda a, b: pl.BlockSpec((a, b), lambda i: (0, 0))
    return pl.pallas_call(
        _t2_body,
        grid=(GRID,),
        in_specs=[pspec, cspec, rspec, bspec, pspec, cspec, rspec, bspec,
                  batspec, full(NG, 3 * H), full(4 * H, H), full(1, H),
                  full(H, H // 2), full(1, H // 2), full(H // 2, NCLS),
                  full(1, NCLS)],
        out_specs=full(NG, NCLS),
        out_shape=jax.ShapeDtypeStruct((NG, NCLS), jnp.float32),
        scratch_shapes=[pltpu.VMEM((NG, 2 * H), jnp.float32)],
    )(ps, cs, rs, bs, pf, cf, rf, bf, batch3, pool1, w1, b1, w2, b2, w3, b3)


def _pad_edges(idx, fill):
    pad = jnp.full((NW * EPW_PAD - E,), fill, idx.dtype)
    return jnp.concatenate([idx, pad]).reshape(NW * CPW, CHUNK)


def kernel(x, edge_index, batch, fc_x, fc_edge_index, sc1_Wrel, sc1_brel,
           sc1_Wroot, sc2_Wrel, sc2_brel, sc2_Wroot, fc1_Wrel, fc1_brel,
           fc1_Wroot, fc2_Wrel, fc2_brel, fc2_Wroot, lin1_W, lin1_b, lin2_W,
           lin2_b, lin3_W, lin3_b):
    src_s = _pad_edges(edge_index[0], 0)
    dst_s = _pad_edges(edge_index[1], N)
    src_f = _pad_edges(fc_edge_index[0], 0)
    dst_f = _pad_edges(fc_edge_index[1], N)
    zeros_h = jnp.zeros((ZCH, H), jnp.float32)
    zcnt_h = jnp.zeros((ZCH, CNTW), jnp.float32)
    ones_h = jnp.ones((SCE, CNTW), jnp.float32)
    bp = jnp.stack([batch[0::2].reshape(GRID, BLK2),
                    batch[1::2].reshape(GRID, BLK2)])
    batch4 = bp.reshape(2, GRID, 1, BLK2)
    dup = lambda b: jnp.concatenate([b, b]).reshape(1, 2 * H)
    b1s, b2s = dup(sc1_brel), dup(sc2_brel)
    b1f, b2f = dup(fc1_brel), dup(fc2_brel)

    xp = x.reshape(N // 2, 2 * D)
    fxp = fc_x.reshape(N // 2, 2 * D)
    y1sp, r1sp, y1fp, r1fp = _t0(xp, fxp, sc1_Wrel, sc1_Wroot,
                                 fc1_Wrel, fc1_Wroot)

    s1 = _make_sc_kernel(True)
    p1s, p1f, cnt_s, cnt_f = s1(y1sp.reshape(N, H), y1fp.reshape(N, H),
                                src_s, dst_s, src_f, dst_f,
                                zeros_h, zcnt_h, ones_h)

    pair = lambda p: p.reshape(NCORE, NPAD // 2, 2 * H)
    cpair = lambda c: c.reshape(NCORE, NPAD // 2, 2 * CNTW)
    cnt_sp, cnt_fp = cpair(cnt_s), cpair(cnt_f)
    y2sp, r2sp, y2fp, r2fp, pool1 = _t1(
        pair(p1s), cnt_sp, r1sp, b1s, sc2_Wrel, sc2_Wroot,
        pair(p1f), cnt_fp, r1fp, b1f, fc2_Wrel, fc2_Wroot, batch4)

    s2 = _make_sc_kernel(False)
    p2s, p2f = s2(y2sp.reshape(N, H), y2fp.reshape(N, H),
                  src_s, dst_s, src_f, dst_f, zeros_h)

    return _t2(pair(p2s), cnt_sp, r2sp, b2s, pair(p2f), cnt_fp, r2fp, b2f,
               batch4, pool1,
               lin1_W, lin1_b.reshape(1, H), lin2_W,
               lin2_b.reshape(1, H // 2), lin3_W, lin3_b.reshape(1, NCLS))


# speedup vs baseline: 1.0307x; 1.0307x over previous
import functools

import jax
import jax.numpy as jnp
from jax import lax
from jax.experimental import pallas as pl
from jax.experimental.pallas import tpu as pltpu
from jax.experimental.pallas import tpu_sc as plsc

N = 10000
E = 320000
D = 128
H = 64
NG = 32
NCLS = 10

NCORE = 2
NSUB = 16
NW = NCORE * NSUB
EPW = E // NW
CHUNK = 256
CPW = 40
SCE = CHUNK
NSC = CPW
NPASS = 4
RPP = CPW // NPASS
NBUF = 2
PD = 1
YRT = N // NSUB
EPW_PAD = CPW * CHUNK
NPAD = 10240
RPT = NPAD // NSUB
ZCH = 128
CNTW = 8

BLK = 2000
BLK2 = BLK // 2
GRID = N // BLK


def _sc_body(with_counts, *refs):
    r = list(refs)
    if with_counts:
        (y_s, y_f, src_s, dst_s, src_f, dst_f, zeros_h, zcnt_h, ones_h,
         out_s, out_f, ocnt_s, ocnt_f, acc, cnt, y_sh) = r[:16]
        r = r[16:]
    else:
        (y_s, y_f, src_s, dst_s, src_f, dst_f, zeros_h,
         out_s, out_f, acc, y_sh) = r[:11]
        r = r[11:]
        cnt = ocnt_s = ocnt_f = None
        zcnt_h = ones_h = None
    src_v, dst_v = r[0], r[1]
    bufs = tuple(r[2:2 + NBUF])
    r = r[2 + NBUF:]
    if with_counts:
        ones_v = r[0]
        r = r[1:]
    else:
        ones_v = None
    gsems = tuple(r[:NBUF])
    ssems = tuple(r[NBUF:2 * NBUF])

    c = lax.axis_index("c")
    s = lax.axis_index("s")
    w = c * NSUB + s

    if with_counts:
        pltpu.sync_copy(ones_h, ones_v)

    branches = [(y_s, src_s, dst_s, out_s, ocnt_s if with_counts else None),
                (y_f, src_f, dst_f, out_f, ocnt_f if with_counts else None)]
    for y_h, src_h, dst_h, out_h, ocnt_h in branches:
        pltpu.sync_copy(y_h.at[pl.ds(s * YRT, YRT)],
                        y_sh.at[pl.ds(s * YRT, YRT)])
        for k in range(RPT // ZCH):
            r0 = s * RPT + k * ZCH
            pltpu.sync_copy(zeros_h, acc.at[pl.ds(r0, ZCH)])
            if with_counts:
                pltpu.sync_copy(zcnt_h, cnt.at[pl.ds(r0, ZCH)])
        plsc.subcore_barrier()

        def gstart(g, i):
            pltpu.async_copy(y_sh.at[src_v.at[g]], bufs[i], gsems[i])

        def gwait(g, i):
            pltpu.make_async_copy(y_sh.at[src_v.at[g]], bufs[i],
                                  gsems[i]).wait()

        def sstart(g, i):
            pltpu.async_copy(bufs[i], acc.at[dst_v.at[g]], ssems[i],
                             add=True)
            if with_counts:
                pltpu.async_copy(ones_v, cnt.at[dst_v.at[g]], ssems[i],
                                 add=True)

        def swait(g, i):
            pltpu.make_async_copy(bufs[i], acc.at[dst_v.at[g]],
                                  ssems[i]).wait()
            if with_counts:
                pltpu.make_async_copy(ones_v, cnt.at[dst_v.at[g]],
                                      ssems[i]).wait()

        for p in range(NPASS):
            base = w * CPW + p * RPP
            pltpu.sync_copy(src_h.at[pl.ds(base, RPP)], src_v)
            pltpu.sync_copy(dst_h.at[pl.ds(base, RPP)], dst_v)

            for k in range(PD):
                gstart(k, k % NBUF)

            def quad(q, carry):
                for i in range(NBUF):
                    g = NBUF * q + i
                    gwait(g, i)
                    sstart(g, i)
                    j = (i + PD) % NBUF

                    @pl.when(g >= PD)
                    def _():
                        swait(g - PD, j)

                    @pl.when(g + PD < RPP)
                    def _():
                        gstart(g + PD, j)
                return carry

            lax.fori_loop(0, RPP // NBUF, quad, 0)
            for t in range(PD):
                g = RPP - PD + t
                swait(g, g % NBUF)
        plsc.subcore_barrier()

        r0 = s * RPT
        pltpu.sync_copy(acc.at[pl.ds(r0, RPT)], out_h.at[c, pl.ds(r0, RPT)])
        if with_counts:
            pltpu.sync_copy(cnt.at[pl.ds(r0, RPT)],
                            ocnt_h.at[c, pl.ds(r0, RPT)])


def _make_sc_kernel(with_counts):
    out_type = [jax.ShapeDtypeStruct((NCORE, NPAD, H), jnp.float32),
                jax.ShapeDtypeStruct((NCORE, NPAD, H), jnp.float32)]
    scratch = [pltpu.VMEM_SHARED((NPAD, H), jnp.float32)]
    if with_counts:
        out_type += [jax.ShapeDtypeStruct((NCORE, NPAD, CNTW), jnp.float32),
                     jax.ShapeDtypeStruct((NCORE, NPAD, CNTW), jnp.float32)]
        scratch += [pltpu.VMEM_SHARED((NPAD, CNTW), jnp.float32)]
    scratch += [pltpu.VMEM_SHARED((N, H), jnp.float32)]
    scratch += [pltpu.VMEM((RPP, CHUNK), jnp.int32),
                pltpu.VMEM((RPP, CHUNK), jnp.int32)]
    scratch += [pltpu.VMEM((SCE, H), jnp.float32)] * NBUF
    if with_counts:
        scratch += [pltpu.VMEM((SCE, CNTW), jnp.float32)]
    scratch += [pltpu.SemaphoreType.DMA] * (2 * NBUF)
    mesh = plsc.VectorSubcoreMesh(core_axis_name="c", subcore_axis_name="s")
    return pl.kernel(functools.partial(_sc_body, with_counts),
                     out_type=out_type, mesh=mesh, scratch_types=scratch,
                     compiler_params=pltpu.CompilerParams(
                         use_tc_tiling_on_sc=False))


def _pair_mm(xe, xo, w):
    return jnp.concatenate(
        [jnp.dot(xe, w, preferred_element_type=jnp.float32),
         jnp.dot(xo, w, preferred_element_type=jnp.float32)], axis=1)


def _t0_body(xp, fxp, ws_rel, ws_root, wf_rel, wf_root,
             y1s, r1s, y1f, r1f):
    xb = xp[...]
    fb = fxp[...]
    xe, xo = xb[:, :D], xb[:, D:]
    fe, fo = fb[:, :D], fb[:, D:]
    y1s[...] = _pair_mm(xe, xo, ws_rel[...])
    r1s[...] = _pair_mm(xe, xo, ws_root[...])
    y1f[...] = _pair_mm(fe, fo, wf_rel[...])
    r1f[...] = _pair_mm(fe, fo, wf_root[...])


def _t0(xp, fxp, ws_rel, ws_root, wf_rel, wf_root):
    blk = pl.BlockSpec((BLK2, 2 * D), lambda i: (i, 0))
    wspec = pl.BlockSpec((D, H), lambda i: (0, 0))
    ospec = pl.BlockSpec((BLK2, 2 * H), lambda i: (i, 0))
    return pl.pallas_call(
        _t0_body,
        grid=(GRID,),
        in_specs=[blk, blk, wspec, wspec, wspec, wspec],
        out_specs=[ospec] * 4,
        out_shape=[jax.ShapeDtypeStruct((N // 2, 2 * H), jnp.float32)] * 4,
    )(xp, fxp, ws_rel, ws_root, wf_rel, wf_root)


def _conv_out(parts, cnt, r, b):
    tot = parts[0] + parts[1]
    c2 = cnt[0] + cnt[1]
    ce = jnp.maximum(c2[:, 0:1], 1.0)
    co = jnp.maximum(c2[:, CNTW:CNTW + 1], 1.0)
    div = jnp.concatenate([jnp.broadcast_to(ce, (BLK2, H)),
                           jnp.broadcast_to(co, (BLK2, H))], axis=1)
    return jax.nn.relu(tot / div + b[...] + r[...])


def _onehot(bids):
    gids = lax.broadcasted_iota(jnp.int32, (BLK2, NG), 1)
    return (bids[:, None] == gids).astype(jnp.float32)


def _pool_contrib(batch_ref, cols_e, cols_o):
    ohe = _onehot(batch_ref[0, 0, 0, :])
    oho = _onehot(batch_ref[1, 0, 0, :])
    dims = (((0,), (0,)), ((), ()))
    aug_e = jnp.concatenate(cols_e, axis=1)
    aug_o = jnp.concatenate(cols_o, axis=1)
    return (lax.dot_general(ohe, aug_e, dims,
                            preferred_element_type=jnp.float32)
            + lax.dot_general(oho, aug_o, dims,
                              preferred_element_type=jnp.float32))


def _t1_body(ps, cs, rs, bs, w2s_rel, w2s_root,
             pf, cf, rf, bf, w2f_rel, w2f_root, batch,
             y2s, r2s, y2f, r2f, pool1):
    i = pl.program_id(0)
    hs = _conv_out(ps[...], cs[...], rs, bs)
    hf = _conv_out(pf[...], cf[...], rf, bf)
    hse, hso = hs[:, :H], hs[:, H:]
    hfe, hfo = hf[:, :H], hf[:, H:]
    y2s[...] = _pair_mm(hse, hso, w2s_rel[...])
    r2s[...] = _pair_mm(hse, hso, w2s_root[...])
    y2f[...] = _pair_mm(hfe, hfo, w2f_rel[...])
    r2f[...] = _pair_mm(hfe, hfo, w2f_root[...])
    ones = jnp.ones((BLK2, H), jnp.float32)
    contrib = _pool_contrib(batch, [hse, hfe, ones], [hso, hfo, ones])

    @pl.when(i == 0)
    def _():
        pool1[...] = jnp.zeros_like(pool1)

    pool1[...] += contrib


def _t1(ps, cs, rs, bs, w2s_rel, w2s_root, pf, cf, rf, bf, w2f_rel, w2f_root,
        batch3):
    pspec = pl.BlockSpec((NCORE, BLK2, 2 * H), lambda i: (0, i, 0))
    cspec = pl.BlockSpec((NCORE, BLK2, 2 * CNTW), lambda i: (0, i, 0))
    rspec = pl.BlockSpec((BLK2, 2 * H), lambda i: (i, 0))
    bspec = pl.BlockSpec((1, 2 * H), lambda i: (0, 0))
    wspec = pl.BlockSpec((H, H), lambda i: (0, 0))
    ospec = pl.BlockSpec((BLK2, 2 * H), lambda i: (i, 0))
    batspec = pl.BlockSpec((2, 1, 1, BLK2), lambda i: (0, i, 0, 0))
    pool_spec = pl.BlockSpec((NG, 3 * H), lambda i: (0, 0))
    return pl.pallas_call(
        _t1_body,
        grid=(GRID,),
        in_specs=[pspec, cspec, rspec, bspec, wspec, wspec,
                  pspec, cspec, rspec, bspec, wspec, wspec, batspec],
        out_specs=[ospec] * 4 + [pool_spec],
        out_shape=[jax.ShapeDtypeStruct((N // 2, 2 * H), jnp.float32)] * 4
        + [jax.ShapeDtypeStruct((NG, 3 * H), jnp.float32)],
    )(ps, cs, rs, bs, w2s_rel, w2s_root, pf, cf, rf, bf, w2f_rel, w2f_root,
      batch3)


def _t2_body(ps, cs, rs, bs, pf, cf, rf, bf, batch, pool1,
             w1, b1, w2, b2, w3, b3, out, pool2):
    i = pl.program_id(0)
    h2s = _conv_out(ps[...], cs[...], rs, bs)
    h2f = _conv_out(pf[...], cf[...], rf, bf)
    contrib = _pool_contrib(batch, [h2s[:, :H], h2f[:, :H]],
                            [h2s[:, H:], h2f[:, H:]])

    @pl.when(i == 0)
    def _():
        pool2[...] = jnp.zeros_like(pool2)

    pool2[...] += contrib

    @pl.when(i == GRID - 1)
    def _():
        p1 = pool1[...]
        p2 = pool2[...]
        gc = jnp.maximum(p1[:, 2 * H:2 * H + 1], 1.0)
        z = jnp.concatenate([p1[:, :H] / gc, p2[:, :H] / gc,
                             p1[:, H:2 * H], p2[:, H:2 * H]], axis=1)
        z = jax.nn.relu(
            jnp.dot(z, w1[...], preferred_element_type=jnp.float32) + b1[...])
        z = jax.nn.relu(
            jnp.dot(z, w2[...], preferred_element_type=jnp.float32) + b2[...])
        z = jnp.dot(z, w3[...], preferred_element_type=jnp.float32) + b3[...]
        m = jnp.max(z, axis=1, keepdims=True)
        lse = m + jnp.log(jnp.sum(jnp.exp(z - m), axis=1, keepdims=True))
        out[...] = z - lse


def _t2(ps, cs, rs, bs, pf, cf, rf, bf, batch3, pool1, w1, b1, w2, b2,
        w3, b3):
    pspec = pl.BlockSpec((NCORE, BLK2, 2 * H), lambda i: (0, i, 0))
    cspec = pl.BlockSpec((NCORE, BLK2, 2 * CNTW), lambda i: (0, i, 0))
    rspec = pl.BlockSpec((BLK2, 2 * H), lambda i: (i, 0))
    bspec = pl.BlockSpec((1, 2 * H), lambda i: (0, 0))
    batspec = pl.BlockSpec((2, 1, 1, BLK2), lambda i: (0, i, 0, 0))
    full = lambda a, b: pl.BlockSpec((a, b), lambda i: (0, 0))
    return pl.pallas_call(
        _t2_body,
        grid=(GRID,),
        in_specs=[pspec, cspec, rspec, bspec, pspec, cspec, rspec, bspec,
                  batspec, full(NG, 3 * H), full(4 * H, H), full(1, H),
                  full(H, H // 2), full(1, H // 2), full(H // 2, NCLS),
                  full(1, NCLS)],
        out_specs=full(NG, NCLS),
        out_shape=jax.ShapeDtypeStruct((NG, NCLS), jnp.float32),
        scratch_shapes=[pltpu.VMEM((NG, 2 * H), jnp.float32)],
    )(ps, cs, rs, bs, pf, cf, rf, bf, batch3, pool1, w1, b1, w2, b2, w3, b3)


def _pad_edges(idx, fill):
    pad = jnp.full((NW * EPW_PAD - E,), fill, idx.dtype)
    return jnp.concatenate([idx, pad]).reshape(NW * CPW, CHUNK)


def kernel(x, edge_index, batch, fc_x, fc_edge_index, sc1_Wrel, sc1_brel,
           sc1_Wroot, sc2_Wrel, sc2_brel, sc2_Wroot, fc1_Wrel, fc1_brel,
           fc1_Wroot, fc2_Wrel, fc2_brel, fc2_Wroot, lin1_W, lin1_b, lin2_W,
           lin2_b, lin3_W, lin3_b):
    src_s = _pad_edges(edge_index[0], 0)
    dst_s = _pad_edges(edge_index[1], N)
    src_f = _pad_edges(fc_edge_index[0], 0)
    dst_f = _pad_edges(fc_edge_index[1], N)
    zeros_h = jnp.zeros((ZCH, H), jnp.float32)
    zcnt_h = jnp.zeros((ZCH, CNTW), jnp.float32)
    ones_h = jnp.ones((SCE, CNTW), jnp.float32)
    bp = jnp.stack([batch[0::2].reshape(GRID, BLK2),
                    batch[1::2].reshape(GRID, BLK2)])
    batch4 = bp.reshape(2, GRID, 1, BLK2)
    dup = lambda b: jnp.concatenate([b, b]).reshape(1, 2 * H)
    b1s, b2s = dup(sc1_brel), dup(sc2_brel)
    b1f, b2f = dup(fc1_brel), dup(fc2_brel)

    xp = x.reshape(N // 2, 2 * D)
    fxp = fc_x.reshape(N // 2, 2 * D)
    y1sp, r1sp, y1fp, r1fp = _t0(xp, fxp, sc1_Wrel, sc1_Wroot,
                                 fc1_Wrel, fc1_Wroot)

    s1 = _make_sc_kernel(True)
    p1s, p1f, cnt_s, cnt_f = s1(y1sp.reshape(N, H), y1fp.reshape(N, H),
                                src_s, dst_s, src_f, dst_f,
                                zeros_h, zcnt_h, ones_h)

    pair = lambda p: p.reshape(NCORE, NPAD // 2, 2 * H)
    cpair = lambda c: c.reshape(NCORE, NPAD // 2, 2 * CNTW)
    cnt_sp, cnt_fp = cpair(cnt_s), cpair(cnt_f)
    y2sp, r2sp, y2fp, r2fp, pool1 = _t1(
        pair(p1s), cnt_sp, r1sp, b1s, sc2_Wrel, sc2_Wroot,
        pair(p1f), cnt_fp, r1fp, b1f, fc2_Wrel, fc2_Wroot, batch4)

    s2 = _make_sc_kernel(False)
    p2s, p2f = s2(y2sp.reshape(N, H), y2fp.reshape(N, H),
                  src_s, dst_s, src_f, dst_f, zeros_h)

    return _t2(pair(p2s), cnt_sp, r2sp, b2s, pair(p2f), cnt_fp, r2fp, b2f,
               batch4, pool1,
               lin1_W, lin1_b.reshape(1, H), lin2_W,
               lin2_b.reshape(1, H // 2), lin3_W, lin3_b.reshape(1, NCLS))
